# baseline (speedup 1.0000x reference)
import functools

import jax
import jax.numpy as jnp
from jax import lax
from jax.experimental import pallas as pl
from jax.experimental.pallas import tpu as pltpu
from jax.experimental.pallas import tpu_sc as plsc

f32 = jnp.float32
i32 = jnp.int32


def _sc_info():
    try:
        info = plsc.get_sparse_core_info()
        return info.num_cores, info.num_subcores
    except Exception:
        return 2, 16


def _sc_gather_small(rvh, Wu, bs, bo, r_idx, u_idx, v_idx):
    B = r_idx.shape[0]
    D = rvh.shape[1]
    NC, NS = _sc_info()
    NW = NC * NS
    bpw = B // NW
    mesh = plsc.VectorSubcoreMesh(core_axis_name="c", subcore_axis_name="s")

    @functools.partial(
        pl.kernel,
        mesh=mesh,
        compiler_params=pltpu.CompilerParams(use_tc_tiling_on_sc=False),
        out_type=(
            jax.ShapeDtypeStruct((B, D), f32),
            jax.ShapeDtypeStruct((B, D), f32),
            jax.ShapeDtypeStruct((B,), f32),
            jax.ShapeDtypeStruct((B,), f32),
        ),
        scratch_types=[
            pltpu.VMEM((bpw,), i32),
            pltpu.VMEM((bpw,), i32),
            pltpu.VMEM((bpw,), i32),
            pltpu.VMEM((bpw, D), f32),
            pltpu.VMEM((bpw, D), f32),
            pltpu.VMEM((bpw,), f32),
            pltpu.VMEM((bpw,), f32),
            pltpu.SemaphoreType.DMA,
        ],
    )
    def ka(rvh_h, Wu_h, bs_h, bo_h, ri_h, ui_h, vi_h,
           Ru_o, rv_o, bsu_o, bov_o,
           ri_v, ui_v, vi_v, Ru_v, rv_v, bsu_v, bov_v, sem):
        wid = lax.axis_index("s") * NC + lax.axis_index("c")
        base = wid * bpw
        pltpu.sync_copy(ri_h.at[pl.ds(base, bpw)], ri_v)
        pltpu.sync_copy(ui_h.at[pl.ds(base, bpw)], ui_v)
        pltpu.sync_copy(vi_h.at[pl.ds(base, bpw)], vi_v)
        copies = [
            pltpu.async_copy(Wu_h.at[ri_v], Ru_v, sem),
            pltpu.async_copy(rvh_h.at[ri_v], rv_v, sem),
            pltpu.async_copy(bs_h.at[ui_v], bsu_v, sem),
            pltpu.async_copy(bo_h.at[vi_v], bov_v, sem),
        ]
        for c in copies:
            c.wait()
        pltpu.sync_copy(Ru_v, Ru_o.at[pl.ds(base, bpw)])
        pltpu.sync_copy(rv_v, rv_o.at[pl.ds(base, bpw)])
        pltpu.sync_copy(bsu_v, bsu_o.at[pl.ds(base, bpw)])
        pltpu.sync_copy(bov_v, bov_o.at[pl.ds(base, bpw)])

    return ka(rvh, Wu, bs, bo, r_idx, u_idx, v_idx)


def _sc_gather_entity(EhT, u_idx, v_idx):
    D, N = EhT.shape
    B = u_idx.shape[0]
    NC, NS = _sc_info()
    NW = NC * NS
    bpw = B // NW
    mesh = plsc.VectorSubcoreMesh(core_axis_name="c", subcore_axis_name="s")

    @functools.partial(
        pl.kernel,
        mesh=mesh,
        compiler_params=pltpu.CompilerParams(use_tc_tiling_on_sc=True),
        out_type=(
            jax.ShapeDtypeStruct((D, B), f32),
            jax.ShapeDtypeStruct((D, B), f32),
        ),
        scratch_types=[
            pltpu.VMEM((bpw,), i32),
            pltpu.VMEM((bpw,), i32),
            pltpu.VMEM((D, bpw), f32),
            pltpu.VMEM((D, bpw), f32),
            pltpu.SemaphoreType.DMA,
            pltpu.SemaphoreType.DMA,
        ],
    )
    def kb(EhT_h, ui_h, vi_h, u_o, v_o, ui_v, vi_v, u_cols, v_cols, sem, sem2):
        wid = lax.axis_index("s") * NC + lax.axis_index("c")
        base = wid * bpw
        pltpu.sync_copy(ui_h.at[pl.ds(base, bpw)], ui_v)
        pltpu.sync_copy(vi_h.at[pl.ds(base, bpw)], vi_v)

        def body(g, carry):
            g0 = g * 16
            idx16u = ui_v[pl.ds(g0, 16)]
            idx16v = vi_v[pl.ds(g0, 16)]
            for k in range(16):
                su = idx16u[k]
                sv = idx16v[k]
                pltpu.make_async_copy(
                    EhT_h.at[:, pl.ds(su, 1)],
                    u_cols.at[:, pl.ds(g0 + k, 1)], sem,
                ).start()
                pltpu.make_async_copy(
                    EhT_h.at[:, pl.ds(sv, 1)],
                    v_cols.at[:, pl.ds(g0 + k, 1)], sem2,
                ).start()
            return carry

        lax.fori_loop(0, bpw // 16, body, 0)
        pltpu.make_async_copy(EhT_h.at[:, pl.ds(0, bpw)], u_cols, sem).wait()
        pltpu.make_async_copy(EhT_h.at[:, pl.ds(0, bpw)], v_cols, sem2).wait()
        pltpu.sync_copy(u_cols, u_o.at[:, pl.ds(base, bpw)])
        pltpu.sync_copy(v_cols, v_o.at[:, pl.ds(base, bpw)])

    return kb(EhT, u_idx, v_idx)


def _norm0(x):
    return jnp.sqrt(jnp.sum(x * x, axis=0, keepdims=True))


def _proj0(x):
    n = _norm0(x)
    return jnp.where(n >= 1.0, x / (n - 1e-5), x)


def _artanh(x):
    return 0.5 * jnp.log((1.0 + x) / (1.0 - x))


def _p_sum0(x, y):
    sqx = jnp.clip(jnp.sum(x * x, axis=0, keepdims=True), 0.0, 1.0 - 1e-5)
    sqy = jnp.clip(jnp.sum(y * y, axis=0, keepdims=True), 0.0, 1.0 - 1e-5)
    dxy = jnp.sum(x * y, axis=0, keepdims=True)
    num = (1.0 + 2.0 * dxy + sqy) * x + (1.0 - sqx) * y
    den = 1.0 + 2.0 * dxy + sqx * sqy
    return num / den


def _math_u_body(uT_ref, Ru_ref, o_ref):
    u = _proj0(uT_ref[...])
    RuT = Ru_ref[...].T
    nu = jnp.clip(_norm0(u), 1e-10, 1.0 - 1e-5)
    u_e = _artanh(nu) * u / nu
    u_W = u_e * RuT
    nw = jnp.clip(_norm0(u_W), 1e-10, None)
    u_m = jnp.tanh(nw) * u_W / nw
    o_ref[...] = _proj0(u_m)


def _math_v_body(um_ref, vT_ref, rv_ref, bsu_ref, bov_ref, o_ref):
    v = _proj0(vT_ref[...])
    rvh_e = _proj0(rv_ref[...].T)
    v_m = _proj0(_p_sum0(v, rvh_e))
    d = _p_sum0(-um_ref[...], v_m)
    nd = jnp.clip(_norm0(d), 1e-10, 1.0 - 1e-5)
    sq = (2.0 * _artanh(nd)) ** 2
    o_ref[...] = -lax.squeeze(sq, (0,)) + bsu_ref[...] + bov_ref[...]


def _tc_math(uT, vT, Ru, rvh_e, bsu, bov, interpret=False):
    D, B = uT.shape
    BLK = 2048
    spec_t = pl.BlockSpec((D, BLK), lambda i: (0, i))
    spec_d = pl.BlockSpec((BLK, D), lambda i: (i, 0))
    spec_1 = pl.BlockSpec((BLK,), lambda i: (i,))
    u_m = pl.pallas_call(
        _math_u_body,
        grid=(B // BLK,),
        in_specs=[spec_t, spec_d],
        out_specs=spec_t,
        out_shape=jax.ShapeDtypeStruct((D, B), jnp.float32),
        interpret=interpret,
    )(uT, Ru)
    out = pl.pallas_call(
        _math_v_body,
        grid=(B // BLK,),
        in_specs=[spec_t, spec_t, spec_d, spec_1, spec_1],
        out_specs=spec_1,
        out_shape=jax.ShapeDtypeStruct((B,), jnp.float32),
        interpret=interpret,
    )(u_m, vT, rvh_e, bsu, bov)
    return out


def kernel(Eh, rvh, Wu, bs, bo, u_idx, r_idx, v_idx):
    u_idx = u_idx.astype(i32)
    r_idx = r_idx.astype(i32)
    v_idx = v_idx.astype(i32)
    Ru, rvh_e, bsu, bov = _sc_gather_small(rvh, Wu, bs, bo, r_idx, u_idx, v_idx)
    uT = jnp.take(Eh, u_idx, axis=0).T
    vT = jnp.take(Eh, v_idx, axis=0).T
    return _tc_math(uT, vT, Ru, rvh_e, bsu, bov)


# speedup vs baseline: 1.3842x; 1.3842x over previous
import functools

import jax
import jax.numpy as jnp
from jax import lax
from jax.experimental import pallas as pl
from jax.experimental.pallas import tpu as pltpu
from jax.experimental.pallas import tpu_sc as plsc

f32 = jnp.float32
i32 = jnp.int32


def _sc_info():
    try:
        info = plsc.get_sparse_core_info()
        return info.num_cores, info.num_subcores
    except Exception:
        return 2, 16


def _sc_gather_bias(bs, bo, u_idx, v_idx):
    B = u_idx.shape[0]
    NC, NS = _sc_info()
    NW = NC * NS
    bpw = B // NW
    mesh = plsc.VectorSubcoreMesh(core_axis_name="c", subcore_axis_name="s")

    @functools.partial(
        pl.kernel,
        mesh=mesh,
        compiler_params=pltpu.CompilerParams(use_tc_tiling_on_sc=False),
        out_type=(
            jax.ShapeDtypeStruct((B,), f32),
            jax.ShapeDtypeStruct((B,), f32),
        ),
        scratch_types=[
            pltpu.VMEM((bpw,), i32),
            pltpu.VMEM((bpw,), i32),
            pltpu.VMEM((bpw,), f32),
            pltpu.VMEM((bpw,), f32),
            pltpu.SemaphoreType.DMA,
        ],
    )
    def ka(bs_h, bo_h, ui_h, vi_h, bsu_o, bov_o,
           ui_v, vi_v, bsu_v, bov_v, sem):
        wid = lax.axis_index("s") * NC + lax.axis_index("c")
        base = wid * bpw
        pltpu.sync_copy(ui_h.at[pl.ds(base, bpw)], ui_v)
        pltpu.sync_copy(vi_h.at[pl.ds(base, bpw)], vi_v)
        copies = [
            pltpu.async_copy(bs_h.at[ui_v], bsu_v, sem),
            pltpu.async_copy(bo_h.at[vi_v], bov_v, sem),
        ]
        for c in copies:
            c.wait()
        pltpu.sync_copy(bsu_v, bsu_o.at[pl.ds(base, bpw)])
        pltpu.sync_copy(bov_v, bov_o.at[pl.ds(base, bpw)])

    return ka(bs, bo, u_idx, v_idx)


def _norm0(x):
    return jnp.sqrt(jnp.sum(x * x, axis=0, keepdims=True))


def _proj0(x):
    n = _norm0(x)
    return jnp.where(n >= 1.0, x / (n - 1e-5), x)


def _artanh(x):
    return 0.5 * jnp.log((1.0 + x) / (1.0 - x))


def _p_sum0(x, y):
    sqx = jnp.clip(jnp.sum(x * x, axis=0, keepdims=True), 0.0, 1.0 - 1e-5)
    sqy = jnp.clip(jnp.sum(y * y, axis=0, keepdims=True), 0.0, 1.0 - 1e-5)
    dxy = jnp.sum(x * y, axis=0, keepdims=True)
    num = (1.0 + 2.0 * dxy + sqy) * x + (1.0 - sqx) * y
    den = 1.0 + 2.0 * dxy + sqx * sqy
    return num / den


def _sel_t(tabT_ref, r):
    nrel = tabT_ref.shape[1]
    onehot = jnp.equal(
        lax.broadcasted_iota(i32, (nrel, 1), 0), r[None, :]
    ).astype(f32)
    return jnp.dot(tabT_ref[...], onehot, preferred_element_type=f32)


def _math_u_body(uT_ref, WuT_ref, r_ref, o_ref):
    u = _proj0(uT_ref[...])
    RuT = _sel_t(WuT_ref, r_ref[...])
    nu = jnp.clip(_norm0(u), 1e-10, 1.0 - 1e-5)
    u_e = _artanh(nu) * u / nu
    u_W = u_e * RuT
    nw = jnp.clip(_norm0(u_W), 1e-10, None)
    u_m = jnp.tanh(nw) * u_W / nw
    o_ref[...] = _proj0(u_m)


def _math_v_body(um_ref, vT_ref, rvhT_ref, r_ref, o_ref):
    v = _proj0(vT_ref[...])
    rvh_e = _proj0(_sel_t(rvhT_ref, r_ref[...]))
    v_m = _proj0(_p_sum0(v, rvh_e))
    d = _p_sum0(-um_ref[...], v_m)
    nd = jnp.clip(_norm0(d), 1e-10, 1.0 - 1e-5)
    sq = (2.0 * _artanh(nd)) ** 2
    o_ref[...] = -lax.squeeze(sq, (0,))


def _tc_math(uT, vT, WuT, rvhT, r_idx, interpret=False):
    D, B = uT.shape
    nrel = WuT.shape[1]
    BLK = 2048
    spec_t = pl.BlockSpec((D, BLK), lambda i: (0, i))
    spec_w = pl.BlockSpec((D, nrel), lambda i: (0, 0))
    spec_1 = pl.BlockSpec((BLK,), lambda i: (i,))
    u_m = pl.pallas_call(
        _math_u_body,
        grid=(B // BLK,),
        in_specs=[spec_t, spec_w, spec_1],
        out_specs=spec_t,
        out_shape=jax.ShapeDtypeStruct((D, B), f32),
        interpret=interpret,
    )(uT, WuT, r_idx)
    out = pl.pallas_call(
        _math_v_body,
        grid=(B // BLK,),
        in_specs=[spec_t, spec_t, spec_w, spec_1],
        out_specs=spec_1,
        out_shape=jax.ShapeDtypeStruct((B,), f32),
        interpret=interpret,
    )(u_m, vT, rvhT, r_idx)
    return out


def kernel(Eh, rvh, Wu, bs, bo, u_idx, r_idx, v_idx):
    u_idx = u_idx.astype(i32)
    r_idx = r_idx.astype(i32)
    v_idx = v_idx.astype(i32)
    bsu, bov = _sc_gather_bias(bs, bo, u_idx, v_idx)
    uT = jnp.take(Eh, u_idx, axis=0).T
    vT = jnp.take(Eh, v_idx, axis=0).T
    neg_sq = _tc_math(uT, vT, Wu.T, rvh.T, r_idx)
    return neg_sq + bsu + bov
